# baseline (speedup 1.0000x reference)
import functools

import jax
import jax.numpy as jnp
from jax import lax
from jax.experimental import pallas as pl
from jax.experimental.pallas import tpu as pltpu
from jax.experimental.pallas import tpu_sc as plsc

N_NODES = 100000
NUM_TYPES = 128

NC = 2
NS = 16
NW = NC * NS

RPW = 2800
N_SC = NW * RPW
C = 400
NCH = RPW // C
GROUPS = C // 16

TC_ROWS = N_NODES - N_SC
TC_BR = 400
TC_B0 = N_SC // TC_BR
TC_NB = TC_ROWS // TC_BR

_mesh = plsc.VectorSubcoreMesh(core_axis_name="c", subcore_axis_name="s")


@functools.partial(
    pl.kernel,
    mesh=_mesh,
    compiler_params=pltpu.CompilerParams(
        needs_layout_passes=False,
        skip_device_barrier=True,
        disable_bounds_checks=True,
        disable_semaphore_checks=True,
    ),
    out_type=jax.ShapeDtypeStruct((N_NODES, NUM_TYPES), jnp.float32),
    scratch_types=[
        pltpu.VMEM((RPW,), jnp.int32),
        pltpu.VMEM((C, NUM_TYPES), jnp.float32),
        pltpu.VMEM((C, NUM_TYPES), jnp.float32),
        pltpu.SemaphoreType.DMA,
        pltpu.SemaphoreType.DMA,
    ],
)
def _onehot_sc(a_hbm, out_hbm, idx_v, buf0, buf1, sem0, sem1):
    wid = lax.axis_index("s") * NC + lax.axis_index("c")
    base = wid * RPW

    pltpu.make_async_copy(a_hbm.at[pl.ds(base, RPW)], idx_v, sem0).start()

    iota16 = lax.iota(jnp.int32, 16)
    ones16 = jnp.ones((16,), jnp.float32)
    zeros16 = jnp.zeros((16,), jnp.float32)
    bufs = (buf0, buf1)
    sems = (sem0, sem1)

    def zrow(r, carry):
        for k in range(NUM_TYPES // 16):
            buf0[r, pl.ds(k * 16, 16)] = zeros16
            buf1[r, pl.ds(k * 16, 16)] = zeros16
        return carry

    lax.fori_loop(0, C, zrow, 0)
    pltpu.make_async_copy(a_hbm.at[pl.ds(base, RPW)], idx_v, sem0).wait()

    def scat(buf, ci, vals):
        def g_body(g, carry):
            a = idx_v[pl.ds(ci * C + g * 16, 16)]
            rows = g * 16 + iota16
            plsc.store_scatter(buf, [rows, a], vals)
            return carry

        lax.fori_loop(0, GROUPS, g_body, 0)

    for ci in range(NCH):
        b = ci % 2
        buf = bufs[b]
        sem = sems[b]
        row0 = base + ci * C
        if ci >= 2:
            pltpu.make_async_copy(buf, out_hbm.at[pl.ds(row0, C)], sem).wait()
            scat(buf, ci - 2, zeros16)
        scat(buf, ci, ones16)
        pltpu.make_async_copy(buf, out_hbm.at[pl.ds(row0, C)], sem).start()

    pltpu.make_async_copy(buf0, out_hbm.at[pl.ds(0, C)], sem0).wait()
    pltpu.make_async_copy(buf1, out_hbm.at[pl.ds(0, C)], sem1).wait()


def _tc_body(sc_ref, a_ref, out_ref):
    del sc_ref
    iota = lax.broadcasted_iota(jnp.int32, (TC_BR, NUM_TYPES), 1)
    out_ref[...] = (a_ref[...] == iota).astype(jnp.float32)


_tc_fill = pl.pallas_call(
    _tc_body,
    grid=(TC_NB,),
    in_specs=[
        pl.BlockSpec(memory_space=pl.ANY),
        pl.BlockSpec((TC_BR, 1), lambda i: (TC_B0 + i, 0)),
    ],
    out_specs=pl.BlockSpec((TC_BR, NUM_TYPES), lambda i: (TC_B0 + i, 0)),
    out_shape=jax.ShapeDtypeStruct((N_NODES, NUM_TYPES), jnp.float32),
    input_output_aliases={0: 0},
)


@jax.jit
def kernel(atomic_numbers, positions):
    del positions
    sc_out = _onehot_sc(atomic_numbers)
    return _tc_fill(sc_out, atomic_numbers.reshape(-1, 1))


# speedup vs baseline: 2.1020x; 2.1020x over previous
import functools

import jax
import jax.numpy as jnp
from jax import lax
from jax.experimental import pallas as pl
from jax.experimental.pallas import tpu as pltpu
from jax.experimental.pallas import tpu_sc as plsc

N_NODES = 100000
NUM_TYPES = 128

NC = 2
NS = 16
NW = NC * NS

RPW = 3200
LAST_BASE = (NW - 1) * RPW
LAST_ROWS = N_NODES - LAST_BASE
C = 200
GROUPS = C // 16

_mesh = plsc.VectorSubcoreMesh(core_axis_name="c", subcore_axis_name="s")


@functools.partial(
    pl.kernel,
    mesh=_mesh,
    compiler_params=pltpu.CompilerParams(
        needs_layout_passes=False,
        skip_device_barrier=True,
        disable_bounds_checks=True,
        disable_semaphore_checks=True,
    ),
    out_type=jax.ShapeDtypeStruct((N_NODES, NUM_TYPES), jnp.float32),
    scratch_types=[
        pltpu.VMEM((RPW,), jnp.int32),
        pltpu.VMEM((C, NUM_TYPES), jnp.float32),
        pltpu.VMEM((C, NUM_TYPES), jnp.float32),
        pltpu.SemaphoreType.DMA,
        pltpu.SemaphoreType.DMA,
    ],
)
def _onehot_sc(a_hbm, out_hbm, idx_v, buf0, buf1, sem0, sem1):
    wid = lax.axis_index("s") * NC + lax.axis_index("c")
    base = wid * RPW
    rows_w = jnp.minimum(RPW, N_NODES - base)
    npairs = rows_w // (2 * C)

    @pl.when(wid < NW - 1)
    def _():
        pltpu.make_async_copy(a_hbm.at[pl.ds(base, RPW)], idx_v, sem0).start()

    @pl.when(wid == NW - 1)
    def _():
        pltpu.make_async_copy(a_hbm.at[pl.ds(LAST_BASE, LAST_ROWS)],
                              idx_v.at[pl.ds(0, LAST_ROWS)], sem0).start()

    iota16 = lax.iota(jnp.int32, 16)
    ones16 = jnp.ones((16,), jnp.float32)
    zeros16 = jnp.zeros((16,), jnp.float32)
    bufs = (buf0, buf1)
    sems = (sem0, sem1)

    def zrow(r, carry):
        for k in range(NUM_TYPES // 16):
            buf0[r, pl.ds(k * 16, 16)] = zeros16
            buf1[r, pl.ds(k * 16, 16)] = zeros16
        return carry

    lax.fori_loop(0, C, zrow, 0)

    @pl.when(wid < NW - 1)
    def _():
        pltpu.make_async_copy(a_hbm.at[pl.ds(base, RPW)], idx_v, sem0).wait()

    @pl.when(wid == NW - 1)
    def _():
        pltpu.make_async_copy(a_hbm.at[pl.ds(LAST_BASE, LAST_ROWS)],
                              idx_v.at[pl.ds(0, LAST_ROWS)], sem0).wait()

    def scat(buf, ci, vals):
        def g_body(g, carry):
            a = idx_v[pl.ds(ci * C + g * 16, 16)]
            rows = g * 16 + iota16
            plsc.store_scatter(buf, [rows, a], vals)
            return carry

        lax.fori_loop(0, GROUPS, g_body, 0)

    def pair(p, carry):
        for b in range(2):
            ci = 2 * p + b
            buf = bufs[b]
            sem = sems[b]
            row0 = base + ci * C

            @pl.when(p > 0)
            def _():
                pltpu.make_async_copy(buf, out_hbm.at[pl.ds(row0, C)], sem).wait()
                scat(buf, ci - 2, zeros16)

            scat(buf, ci, ones16)
            pltpu.make_async_copy(buf, out_hbm.at[pl.ds(row0, C)], sem).start()
        return carry

    lax.fori_loop(0, npairs, pair, 0)

    pltpu.make_async_copy(buf0, out_hbm.at[pl.ds(0, C)], sem0).wait()
    pltpu.make_async_copy(buf1, out_hbm.at[pl.ds(0, C)], sem1).wait()


@jax.jit
def kernel(atomic_numbers, positions):
    del positions
    return _onehot_sc(atomic_numbers)


# speedup vs baseline: 2.1657x; 1.0303x over previous
import functools

import jax
import jax.numpy as jnp
from jax import lax
from jax.experimental import pallas as pl
from jax.experimental.pallas import tpu as pltpu
from jax.experimental.pallas import tpu_sc as plsc

N_NODES = 100000
NUM_TYPES = 128

NC = 2
NS = 16
NW = NC * NS

RPW = 3200
LAST_BASE = (NW - 1) * RPW
LAST_ROWS = N_NODES - LAST_BASE
C = 80
GROUPS = C // 16

_mesh = plsc.VectorSubcoreMesh(core_axis_name="c", subcore_axis_name="s")


@functools.partial(
    pl.kernel,
    mesh=_mesh,
    compiler_params=pltpu.CompilerParams(
        needs_layout_passes=False,
        skip_device_barrier=True,
        disable_bounds_checks=True,
        disable_semaphore_checks=True,
    ),
    out_type=jax.ShapeDtypeStruct((N_NODES, NUM_TYPES), jnp.float32),
    scratch_types=[
        pltpu.VMEM((RPW,), jnp.int32),
        pltpu.VMEM((C, NUM_TYPES), jnp.float32),
        pltpu.VMEM((C, NUM_TYPES), jnp.float32),
        pltpu.SemaphoreType.DMA,
        pltpu.SemaphoreType.DMA,
    ],
)
def _onehot_sc(a_hbm, out_hbm, idx_v, buf0, buf1, sem0, sem1):
    wid = lax.axis_index("s") * NC + lax.axis_index("c")
    base = wid * RPW
    rows_w = jnp.minimum(RPW, N_NODES - base)
    npairs = rows_w // (2 * C)

    @pl.when(wid < NW - 1)
    def _():
        pltpu.make_async_copy(a_hbm.at[pl.ds(base, RPW)], idx_v, sem0).start()

    @pl.when(wid == NW - 1)
    def _():
        pltpu.make_async_copy(a_hbm.at[pl.ds(LAST_BASE, LAST_ROWS)],
                              idx_v.at[pl.ds(0, LAST_ROWS)], sem0).start()

    iota16 = lax.iota(jnp.int32, 16)
    ones16 = jnp.ones((16,), jnp.float32)
    zeros16 = jnp.zeros((16,), jnp.float32)
    bufs = (buf0, buf1)
    sems = (sem0, sem1)

    def zrow(r, carry):
        for k in range(NUM_TYPES // 16):
            buf0[r, pl.ds(k * 16, 16)] = zeros16
            buf1[r, pl.ds(k * 16, 16)] = zeros16
        return carry

    lax.fori_loop(0, C, zrow, 0)

    @pl.when(wid < NW - 1)
    def _():
        pltpu.make_async_copy(a_hbm.at[pl.ds(base, RPW)], idx_v, sem0).wait()

    @pl.when(wid == NW - 1)
    def _():
        pltpu.make_async_copy(a_hbm.at[pl.ds(LAST_BASE, LAST_ROWS)],
                              idx_v.at[pl.ds(0, LAST_ROWS)], sem0).wait()

    def scat(buf, ci, vals):
        def g_body(g, carry):
            a = idx_v[pl.ds(ci * C + g * 16, 16)]
            rows = g * 16 + iota16
            plsc.store_scatter(buf, [rows, a], vals)
            return carry

        lax.fori_loop(0, GROUPS, g_body, 0)

    def pair(p, carry):
        for b in range(2):
            ci = 2 * p + b
            buf = bufs[b]
            sem = sems[b]
            row0 = base + ci * C

            @pl.when(p > 0)
            def _():
                pltpu.make_async_copy(buf, out_hbm.at[pl.ds(row0, C)], sem).wait()
                scat(buf, ci - 2, zeros16)

            scat(buf, ci, ones16)
            pltpu.make_async_copy(buf, out_hbm.at[pl.ds(row0, C)], sem).start()
        return carry

    lax.fori_loop(0, npairs, pair, 0)

    pltpu.make_async_copy(buf0, out_hbm.at[pl.ds(0, C)], sem0).wait()
    pltpu.make_async_copy(buf1, out_hbm.at[pl.ds(0, C)], sem1).wait()


@jax.jit
def kernel(atomic_numbers, positions):
    del positions
    return _onehot_sc(atomic_numbers)


# speedup vs baseline: 2.1694x; 1.0017x over previous
import functools

import jax
import jax.numpy as jnp
from jax import lax
from jax.experimental import pallas as pl
from jax.experimental.pallas import tpu as pltpu
from jax.experimental.pallas import tpu_sc as plsc

N_NODES = 100000
NUM_TYPES = 128

NC = 2
NS = 16
NW = NC * NS

RPW = 3200
LAST_BASE = (NW - 1) * RPW
LAST_ROWS = N_NODES - LAST_BASE
C = 80
GROUPS = C // 16

_mesh = plsc.VectorSubcoreMesh(core_axis_name="c", subcore_axis_name="s")


@functools.partial(
    pl.kernel,
    mesh=_mesh,
    compiler_params=pltpu.CompilerParams(
        needs_layout_passes=False,
        disable_bounds_checks=True,
        disable_semaphore_checks=True,
    ),
    out_type=jax.ShapeDtypeStruct((N_NODES, NUM_TYPES), jnp.float32),
    scratch_types=[
        pltpu.VMEM((RPW,), jnp.int32),
        pltpu.VMEM((C, NUM_TYPES), jnp.float32),
        pltpu.VMEM((C, NUM_TYPES), jnp.float32),
        pltpu.SemaphoreType.DMA,
        pltpu.SemaphoreType.DMA,
    ],
)
def _onehot_sc(a_hbm, out_hbm, idx_v, buf0, buf1, sem0, sem1):
    wid = lax.axis_index("s") * NC + lax.axis_index("c")
    base = wid * RPW
    rows_w = jnp.minimum(RPW, N_NODES - base)
    npairs = rows_w // (2 * C)

    @pl.when(wid < NW - 1)
    def _():
        pltpu.make_async_copy(a_hbm.at[pl.ds(base, RPW)], idx_v, sem0).start()

    @pl.when(wid == NW - 1)
    def _():
        pltpu.make_async_copy(a_hbm.at[pl.ds(LAST_BASE, LAST_ROWS)],
                              idx_v.at[pl.ds(0, LAST_ROWS)], sem0).start()

    iota16 = lax.iota(jnp.int32, 16)
    ones16 = jnp.ones((16,), jnp.float32)
    zeros16 = jnp.zeros((16,), jnp.float32)
    bufs = (buf0, buf1)
    sems = (sem0, sem1)

    def zrow(r, carry):
        for k in range(NUM_TYPES // 16):
            buf0[r, pl.ds(k * 16, 16)] = zeros16
            buf1[r, pl.ds(k * 16, 16)] = zeros16
        return carry

    lax.fori_loop(0, C, zrow, 0)

    @pl.when(wid < NW - 1)
    def _():
        pltpu.make_async_copy(a_hbm.at[pl.ds(base, RPW)], idx_v, sem0).wait()

    @pl.when(wid == NW - 1)
    def _():
        pltpu.make_async_copy(a_hbm.at[pl.ds(LAST_BASE, LAST_ROWS)],
                              idx_v.at[pl.ds(0, LAST_ROWS)], sem0).wait()

    def scat(buf, ci, vals):
        def g_body(g, carry):
            a = idx_v[pl.ds(ci * C + g * 16, 16)]
            rows = g * 16 + iota16
            plsc.store_scatter(buf, [rows, a], vals)
            return carry

        lax.fori_loop(0, GROUPS, g_body, 0)

    def pair(p, carry):
        for b in range(2):
            ci = 2 * p + b
            buf = bufs[b]
            sem = sems[b]
            row0 = base + ci * C

            @pl.when(p > 0)
            def _():
                pltpu.make_async_copy(buf, out_hbm.at[pl.ds(row0, C)], sem).wait()
                scat(buf, ci - 2, zeros16)

            scat(buf, ci, ones16)
            pltpu.make_async_copy(buf, out_hbm.at[pl.ds(row0, C)], sem).start()
        return carry

    lax.fori_loop(0, npairs, pair, 0)

    pltpu.make_async_copy(buf0, out_hbm.at[pl.ds(0, C)], sem0).wait()
    pltpu.make_async_copy(buf1, out_hbm.at[pl.ds(0, C)], sem1).wait()


@jax.jit
def kernel(atomic_numbers, positions):
    del positions
    return _onehot_sc(atomic_numbers)
